# baseline (speedup 1.0000x reference)
import functools

import jax
import jax.numpy as jnp
from jax import lax
from jax.experimental import pallas as pl
from jax.experimental.pallas import tpu as pltpu
from jax.experimental.pallas import tpu_sc as plsc

_N = 10000
_E = 320000
_IN = 128
_HALF = _IN // 2
_HID = 256
_OUT = 128
_G = 16

_CHUNK = 128
_NCHUNK = _E // _CHUNK
_NSUB = 16
_BASE_ITERS = _NCHUNK // _NSUB
_XTRA = _NCHUNK - _BASE_ITERS * _NSUB
_NBUF = 4
_TILES = 16
_RPT = 624
_RTAIL = _N - _TILES * _RPT


def _sc_body(x2_hbm, edge_hbm, batch_hbm,
             acc0_hbm, acc1_hbm, cp0_hbm, cp1_hbm, *sc):
    srcs = sc[0:_NBUF]
    srcs2 = sc[_NBUF:2 * _NBUF]
    dsts = sc[2 * _NBUF:3 * _NBUF]
    gbs = sc[3 * _NBUF:4 * _NBUF]
    rows = sc[4 * _NBUF:5 * _NBUF]
    ohs = sc[5 * _NBUF:5 * _NBUF + _NBUF // 2]
    k = 5 * _NBUF + _NBUF // 2
    cob_v = sc[k]
    acc_sh = sc[k + 1]
    c_sh = sc[k + 2]
    sem_i, sem_g, sem_b, sem_s, sem_c = sc[k + 3:k + 8]

    c = lax.axis_index("c")
    s = lax.axis_index("s")
    t = s
    rbase = pl.multiple_of(t * _RPT, 8)
    tail = pl.multiple_of(_TILES * _RPT, 8)

    zero16 = jnp.zeros((16,), jnp.float32)
    one16 = jnp.ones((16,), jnp.float32)
    iota16 = lax.iota(jnp.int32, 16)

    def zrow(i, carry):
        for j in range(_HALF // 16):
            rows[0][i, pl.ds(j * 16, 16)] = zero16
        return carry
    lax.fori_loop(0, _CHUNK, zrow, 0)

    def zco(i, carry):
        cob_v[i, :] = zero16
        return carry
    lax.fori_loop(0, _RPT, zco, 0)

    for k2 in range(4):
        pltpu.sync_copy(rows[0], acc_sh.at[pl.ds(rbase + k2 * _CHUNK, _CHUNK), :])
    pltpu.sync_copy(rows[0].at[pl.ds(0, _RPT - 512), :],
                    acc_sh.at[pl.ds(rbase + 512, _RPT - 512), :])
    pltpu.sync_copy(cob_v, c_sh.at[pl.ds(rbase, _RPT), :])

    @pl.when(t == 0)
    def _():
        pltpu.sync_copy(rows[0].at[pl.ds(0, _RTAIL), :],
                        acc_sh.at[pl.ds(tail, _RTAIL), :])
        pltpu.sync_copy(cob_v.at[pl.ds(0, _RTAIL), :],
                        c_sh.at[pl.ds(tail, _RTAIL), :])

    plsc.subcore_barrier()

    start = s * _BASE_ITERS + jnp.minimum(s, _XTRA)
    cnt = jnp.where(s < _XTRA, _BASE_ITERS + 1, _BASE_ITERS)
    nfull = cnt // _NBUF

    def build_onehot(b):
        oh = ohs[b // 2]

        def onehot(j, cc):
            gbv = gbs[b][pl.ds(j * 16, 16)]
            for e in range(16):
                bge = jnp.broadcast_to(gbv[e], (16,))
                oh[j * 16 + e, :] = jnp.where(iota16 == bge, one16, zero16)
            return cc
        lax.fori_loop(0, _CHUNK // 16, onehot, 0)

    def counts(slots):
        chs = []
        for b in slots:
            build_onehot(b)
            chs.append(pltpu.async_copy(ohs[b // 2], c_sh.at[srcs[b]], sem_c,
                                        add=True))
        for h in chs:
            h.wait()

    def idx2(b):
        for j in range(_CHUNK // 16):
            sl = pl.ds(j * 16, 16)
            srcs2[b][sl] = srcs[b][sl] * 2 + c

    def body(q, carry):
        g0 = start + q * _NBUF
        ihs = []
        for b in range(_NBUF):
            base = pl.multiple_of((g0 + b) * _CHUNK, 8)
            ihs.append(pltpu.async_copy(edge_hbm.at[0, pl.ds(base, _CHUNK)],
                                        srcs[b], sem_i))
            ihs.append(pltpu.async_copy(edge_hbm.at[1, pl.ds(base, _CHUNK)],
                                        dsts[b], sem_i))
        ghs, bhs = [], []
        for b in range(_NBUF):
            ihs[2 * b].wait()
            ihs[2 * b + 1].wait()
            idx2(b)
            ghs.append(pltpu.async_copy(x2_hbm.at[srcs2[b]], rows[b], sem_g))
            bhs.append(pltpu.async_copy(batch_hbm.at[dsts[b]], gbs[b], sem_b))
        shs = []
        for b in range(_NBUF):
            ghs[b].wait()
            shs.append(pltpu.async_copy(rows[b], acc_sh.at[dsts[b]], sem_s,
                                        add=True))
        for b in range(_NBUF):
            bhs[b].wait()

        @pl.when(c == 0)
        def _():
            counts(range(0, _NBUF, 2))

        @pl.when(c == 1)
        def _():
            counts(range(1, _NBUF, 2))

        for h in shs:
            h.wait()
        return carry

    lax.fori_loop(0, nfull, body, 0)

    def tail_body(i, carry):
        base = pl.multiple_of((start + i) * _CHUNK, 8)
        pltpu.sync_copy(edge_hbm.at[0, pl.ds(base, _CHUNK)], srcs[0])
        pltpu.sync_copy(edge_hbm.at[1, pl.ds(base, _CHUNK)], dsts[0])
        idx2(0)
        pltpu.async_copy(x2_hbm.at[srcs2[0]], rows[0], sem_g).wait()
        pltpu.sync_copy(rows[0], acc_sh.at[dsts[0]], add=True)

        @pl.when(lax.rem(i, 2) == c)
        def _():
            pltpu.async_copy(batch_hbm.at[dsts[0]], gbs[0], sem_b).wait()
            build_onehot(0)
            pltpu.sync_copy(ohs[0], c_sh.at[srcs[0]], add=True)

        return carry

    lax.fori_loop(nfull * _NBUF, cnt, tail_body, 0)

    plsc.subcore_barrier()

    def wout(buf, sl, h0, h1):
        @pl.when(c == 0)
        def _():
            pltpu.sync_copy(buf, h0.at[sl, :])

        @pl.when(c == 1)
        def _():
            pltpu.sync_copy(buf, h1.at[sl, :])

    for k2 in range(4):
        sl = pl.ds(rbase + k2 * _CHUNK, _CHUNK)
        pltpu.sync_copy(acc_sh.at[sl, :], rows[0])
        wout(rows[0], sl, acc0_hbm, acc1_hbm)
    sl = pl.ds(rbase + 512, _RPT - 512)
    pltpu.sync_copy(acc_sh.at[sl, :], rows[0].at[pl.ds(0, _RPT - 512), :])
    wout(rows[0].at[pl.ds(0, _RPT - 512), :], sl, acc0_hbm, acc1_hbm)
    pltpu.sync_copy(c_sh.at[pl.ds(rbase, _RPT), :], cob_v)
    wout(cob_v, pl.ds(rbase, _RPT), cp0_hbm, cp1_hbm)

    @pl.when(t == 0)
    def _():
        sl = pl.ds(tail, _RTAIL)
        pltpu.sync_copy(acc_sh.at[sl, :], rows[0].at[pl.ds(0, _RTAIL), :])
        wout(rows[0].at[pl.ds(0, _RTAIL), :], sl, acc0_hbm, acc1_hbm)
        pltpu.sync_copy(c_sh.at[sl, :], cob_v.at[pl.ds(0, _RTAIL), :])
        wout(cob_v.at[pl.ds(0, _RTAIL), :], sl, cp0_hbm, cp1_hbm)


def _sc_scatter(x2, edge_index, batch):
    mesh = plsc.VectorSubcoreMesh(core_axis_name="c", subcore_axis_name="s")
    scratch = (
        [pltpu.VMEM((_CHUNK,), jnp.int32) for _ in range(_NBUF)]
        + [pltpu.VMEM((_CHUNK,), jnp.int32) for _ in range(_NBUF)]
        + [pltpu.VMEM((_CHUNK,), jnp.int32) for _ in range(_NBUF)]
        + [pltpu.VMEM((_CHUNK,), jnp.int32) for _ in range(_NBUF)]
        + [pltpu.VMEM((_CHUNK, _HALF), jnp.float32) for _ in range(_NBUF)]
        + [pltpu.VMEM((_CHUNK, _G), jnp.float32) for _ in range(_NBUF // 2)]
        + [
            pltpu.VMEM((_RPT, _G), jnp.float32),
            pltpu.VMEM_SHARED((_N, _HALF), jnp.float32),
            pltpu.VMEM_SHARED((_N, _G), jnp.float32),
            pltpu.SemaphoreType.DMA,
            pltpu.SemaphoreType.DMA,
            pltpu.SemaphoreType.DMA,
            pltpu.SemaphoreType.DMA,
            pltpu.SemaphoreType.DMA,
        ]
    )
    fn = pl.kernel(
        _sc_body,
        mesh=mesh,
        compiler_params=pltpu.CompilerParams(use_tc_tiling_on_sc=False),
        out_type=[jax.ShapeDtypeStruct((_N, _HALF), jnp.float32),
                  jax.ShapeDtypeStruct((_N, _HALF), jnp.float32),
                  jax.ShapeDtypeStruct((_N, _G), jnp.float32),
                  jax.ShapeDtypeStruct((_N, _G), jnp.float32)],
        scratch_types=scratch,
    )
    return fn(x2, edge_index, batch)


def _tc_body(p0_r, p1_r, x_r, ct0_r, ct1_r, b3_r,
             w1rel_r, w1root_r, b1_r, w2rel_r, w2root_r, b2_r,
             out_r, C2_r, S2_r, NG_r):
    k = pl.program_id(0)
    nk = pl.num_programs(0)

    @pl.when(k == 0)
    def _():
        C2_r[...] = jnp.zeros_like(C2_r)
        S2_r[...] = jnp.zeros_like(S2_r)
        NG_r[...] = jnp.zeros_like(NG_r)

    dnt = (((1,), (1,)), ((), ()))
    agg = jnp.concatenate([p0_r[...], p1_r[...]], axis=1)
    h = jnp.maximum(
        lax.dot_general(agg, w1rel_r[...], dnt,
                        preferred_element_type=jnp.float32)
        + lax.dot_general(x_r[...], w1root_r[...], dnt,
                          preferred_element_type=jnp.float32)
        + b1_r[...], 0.0)

    ct = ct0_r[...] + ct1_r[...]
    C2_r[...] += lax.dot_general(ct, h, (((0,), (0,)), ((), ())),
                                 preferred_element_type=jnp.float32)

    rows = h.shape[0]
    gids = lax.broadcasted_iota(jnp.int32, (_G, rows), 0)
    mask = (b3_r[...].reshape(1, rows) == gids).astype(jnp.float32)
    S2_r[...] += jnp.dot(mask, h, preferred_element_type=jnp.float32)
    NG_r[...] += jnp.broadcast_to(jnp.sum(mask, axis=1, keepdims=True),
                                  NG_r.shape)

    @pl.when(k == nk - 1)
    def _():
        ng = NG_r[...][:, :1]
        ps = (lax.dot_general(C2_r[...], w2rel_r[...], dnt,
                              preferred_element_type=jnp.float32)
              + lax.dot_general(S2_r[...], w2root_r[...], dnt,
                                preferred_element_type=jnp.float32)
              + ng * b2_r[...])
        pooled = ps / jnp.maximum(ng, 1.0)
        m = jnp.max(pooled, axis=1, keepdims=True)
        sh = pooled - m
        out_r[...] = sh - jnp.log(jnp.sum(jnp.exp(sh), axis=1, keepdims=True))


_ROWS_BLK = 1000
_NBLK = _N // _ROWS_BLK


def _tc_pooled(p0, p1, x, ct0, ct1, batch3,
               w1rel, w1root, b1r, w2rel, w2root, b2r):
    return pl.pallas_call(
        _tc_body,
        grid=(_NBLK,),
        in_specs=[
            pl.BlockSpec((_ROWS_BLK, _HALF), lambda k: (k, 0)),
            pl.BlockSpec((_ROWS_BLK, _HALF), lambda k: (k, 0)),
            pl.BlockSpec((_ROWS_BLK, _IN), lambda k: (k, 0)),
            pl.BlockSpec((_ROWS_BLK, _G), lambda k: (k, 0)),
            pl.BlockSpec((_ROWS_BLK, _G), lambda k: (k, 0)),
            pl.BlockSpec((1, 1, _ROWS_BLK), lambda k: (k, 0, 0)),
            pl.BlockSpec((_HID, _IN), lambda k: (0, 0)),
            pl.BlockSpec((_HID, _IN), lambda k: (0, 0)),
            pl.BlockSpec((1, _HID), lambda k: (0, 0)),
            pl.BlockSpec((_OUT, _HID), lambda k: (0, 0)),
            pl.BlockSpec((_OUT, _HID), lambda k: (0, 0)),
            pl.BlockSpec((1, _OUT), lambda k: (0, 0)),
        ],
        out_specs=pl.BlockSpec((_G, _OUT), lambda k: (0, 0)),
        out_shape=jax.ShapeDtypeStruct((_G, _OUT), jnp.float32),
        scratch_shapes=[
            pltpu.VMEM((_G, _HID), jnp.float32),
            pltpu.VMEM((_G, _HID), jnp.float32),
            pltpu.VMEM((_G, _OUT), jnp.float32),
        ],
    )(p0, p1, x, ct0, ct1, batch3,
      w1rel, w1root, b1r, w2rel, w2root, b2r)


def kernel(x, edge_index, batch, W1_rel, W1_root, b1, W2_rel, W2_root, b2):
    acc0, acc1, cp0, cp1 = _sc_scatter(x.reshape(2 * _N, _HALF),
                                       edge_index, batch)
    batch3 = batch.reshape(_NBLK, 1, _ROWS_BLK)
    return _tc_pooled(
        acc0, acc1, x, cp0, cp1, batch3,
        W1_rel, W1_root, b1.reshape(1, _HID),
        W2_rel, W2_root, b2.reshape(1, _OUT))


# speedup vs baseline: 1.0205x; 1.0205x over previous
import functools

import jax
import jax.numpy as jnp
from jax import lax
from jax.experimental import pallas as pl
from jax.experimental.pallas import tpu as pltpu
from jax.experimental.pallas import tpu_sc as plsc

_N = 10000
_E = 320000
_IN = 128
_HALF = _IN // 2
_HID = 256
_OUT = 128
_G = 16

_CHUNK = 128
_NCHUNK = _E // _CHUNK
_NSUB = 16
_BASE_ITERS = _NCHUNK // _NSUB
_XTRA = _NCHUNK - _BASE_ITERS * _NSUB
_NBUF = 4
_TILES = 16
_RPT = 624
_RTAIL = _N - _TILES * _RPT


def _sc_body(x2_hbm, edge_hbm, batch_hbm,
             acc0_hbm, acc1_hbm, cp0_hbm, cp1_hbm, *sc):
    srcs = sc[0:_NBUF]
    srcs2 = sc[_NBUF:2 * _NBUF]
    dsts = sc[2 * _NBUF:3 * _NBUF]
    gbs = sc[3 * _NBUF:4 * _NBUF]
    rows = sc[4 * _NBUF:5 * _NBUF]
    fidx = sc[5 * _NBUF:5 * _NBUF + _NBUF // 2]
    k = 5 * _NBUF + _NBUF // 2
    ones_v = sc[k]
    cob_v = sc[k + 1]
    acc_sh = sc[k + 2]
    c_sh = sc[k + 3]
    sem_i, sem_g, sem_b, sem_s, sem_c = sc[k + 4:k + 9]

    c = lax.axis_index("c")
    s = lax.axis_index("s")
    t = s
    rbase = pl.multiple_of(t * _RPT, 8)
    tail = pl.multiple_of(_TILES * _RPT, 8)

    fbase = pl.multiple_of(t * _RPT * _G, 128)
    flen = _RPT * _G
    ftail = pl.multiple_of(_TILES * _RPT * _G, 128)

    zero16 = jnp.zeros((16,), jnp.float32)
    one16 = jnp.ones((16,), jnp.float32)

    def zrow(i, carry):
        for j in range(_HALF // 16):
            rows[0][i, pl.ds(j * 16, 16)] = zero16
        return carry
    lax.fori_loop(0, _CHUNK, zrow, 0)

    def zco(i, carry):
        cob_v[pl.ds(i * 16, 16)] = zero16
        return carry
    lax.fori_loop(0, flen // 16, zco, 0)

    for j in range(_CHUNK // 16):
        ones_v[pl.ds(j * 16, 16)] = one16

    for k2 in range(4):
        pltpu.sync_copy(rows[0], acc_sh.at[pl.ds(rbase + k2 * _CHUNK, _CHUNK), :])
    pltpu.sync_copy(rows[0].at[pl.ds(0, _RPT - 512), :],
                    acc_sh.at[pl.ds(rbase + 512, _RPT - 512), :])
    pltpu.sync_copy(cob_v, c_sh.at[pl.ds(fbase, flen)])

    @pl.when(t == 0)
    def _():
        pltpu.sync_copy(rows[0].at[pl.ds(0, _RTAIL), :],
                        acc_sh.at[pl.ds(tail, _RTAIL), :])
        pltpu.sync_copy(cob_v.at[pl.ds(0, _RTAIL * _G)],
                        c_sh.at[pl.ds(ftail, _RTAIL * _G)])

    plsc.subcore_barrier()

    start = s * _BASE_ITERS + jnp.minimum(s, _XTRA)
    cnt = jnp.where(s < _XTRA, _BASE_ITERS + 1, _BASE_ITERS)
    nfull = cnt // _NBUF

    def cidx(b):
        f = fidx[b // 2]
        for j in range(_CHUNK // 16):
            sl = pl.ds(j * 16, 16)
            f[sl] = srcs[b][sl] * _G + gbs[b][sl]

    def counts(slots):
        chs = []
        for b in slots:
            cidx(b)
            chs.append(pltpu.async_copy(ones_v, c_sh.at[fidx[b // 2]], sem_c,
                                        add=True))
        for h in chs:
            h.wait()

    def idx2(b):
        for j in range(_CHUNK // 16):
            sl = pl.ds(j * 16, 16)
            srcs2[b][sl] = srcs[b][sl] * 2 + c

    def body(q, carry):
        g0 = start + q * _NBUF
        ihs = []
        for b in range(_NBUF):
            base = pl.multiple_of((g0 + b) * _CHUNK, 8)
            ihs.append(pltpu.async_copy(edge_hbm.at[0, pl.ds(base, _CHUNK)],
                                        srcs[b], sem_i))
            ihs.append(pltpu.async_copy(edge_hbm.at[1, pl.ds(base, _CHUNK)],
                                        dsts[b], sem_i))
        ghs, bhs = [], []
        for b in range(_NBUF):
            ihs[2 * b].wait()
            ihs[2 * b + 1].wait()
            idx2(b)
            ghs.append(pltpu.async_copy(x2_hbm.at[srcs2[b]], rows[b], sem_g))
            bhs.append(pltpu.async_copy(batch_hbm.at[dsts[b]], gbs[b], sem_b))
        shs = []
        for b in range(_NBUF):
            ghs[b].wait()
            shs.append(pltpu.async_copy(rows[b], acc_sh.at[dsts[b]], sem_s,
                                        add=True))
        for b in range(_NBUF):
            bhs[b].wait()

        @pl.when(c == 0)
        def _():
            counts(range(0, _NBUF, 2))

        @pl.when(c == 1)
        def _():
            counts(range(1, _NBUF, 2))

        for h in shs:
            h.wait()
        return carry

    lax.fori_loop(0, nfull, body, 0)

    def tail_body(i, carry):
        base = pl.multiple_of((start + i) * _CHUNK, 8)
        pltpu.sync_copy(edge_hbm.at[0, pl.ds(base, _CHUNK)], srcs[0])
        pltpu.sync_copy(edge_hbm.at[1, pl.ds(base, _CHUNK)], dsts[0])
        idx2(0)
        pltpu.async_copy(x2_hbm.at[srcs2[0]], rows[0], sem_g).wait()
        pltpu.sync_copy(rows[0], acc_sh.at[dsts[0]], add=True)

        @pl.when(lax.rem(i, 2) == c)
        def _():
            pltpu.async_copy(batch_hbm.at[dsts[0]], gbs[0], sem_b).wait()
            cidx(0)
            pltpu.sync_copy(ones_v, c_sh.at[fidx[0]], add=True)

        return carry

    lax.fori_loop(nfull * _NBUF, cnt, tail_body, 0)

    plsc.subcore_barrier()

    def wout(buf, sl, h0, h1):
        @pl.when(c == 0)
        def _():
            pltpu.sync_copy(buf, h0.at[sl, :])

        @pl.when(c == 1)
        def _():
            pltpu.sync_copy(buf, h1.at[sl, :])

    def wout1(buf, sl, h0, h1):
        @pl.when(c == 0)
        def _():
            pltpu.sync_copy(buf, h0.at[sl])

        @pl.when(c == 1)
        def _():
            pltpu.sync_copy(buf, h1.at[sl])

    for k2 in range(4):
        sl = pl.ds(rbase + k2 * _CHUNK, _CHUNK)
        pltpu.sync_copy(acc_sh.at[sl, :], rows[0])
        wout(rows[0], sl, acc0_hbm, acc1_hbm)
    sl = pl.ds(rbase + 512, _RPT - 512)
    pltpu.sync_copy(acc_sh.at[sl, :], rows[0].at[pl.ds(0, _RPT - 512), :])
    wout(rows[0].at[pl.ds(0, _RPT - 512), :], sl, acc0_hbm, acc1_hbm)
    pltpu.sync_copy(c_sh.at[pl.ds(fbase, flen)], cob_v)
    wout1(cob_v, pl.ds(fbase, flen), cp0_hbm, cp1_hbm)

    @pl.when(t == 0)
    def _():
        sl = pl.ds(tail, _RTAIL)
        pltpu.sync_copy(acc_sh.at[sl, :], rows[0].at[pl.ds(0, _RTAIL), :])
        wout(rows[0].at[pl.ds(0, _RTAIL), :], sl, acc0_hbm, acc1_hbm)
        pltpu.sync_copy(c_sh.at[pl.ds(ftail, _RTAIL * _G)],
                        cob_v.at[pl.ds(0, _RTAIL * _G)])
        wout1(cob_v.at[pl.ds(0, _RTAIL * _G)], pl.ds(ftail, _RTAIL * _G),
              cp0_hbm, cp1_hbm)


def _sc_scatter(x2, edge_index, batch):
    mesh = plsc.VectorSubcoreMesh(core_axis_name="c", subcore_axis_name="s")
    scratch = (
        [pltpu.VMEM((_CHUNK,), jnp.int32) for _ in range(_NBUF)]
        + [pltpu.VMEM((_CHUNK,), jnp.int32) for _ in range(_NBUF)]
        + [pltpu.VMEM((_CHUNK,), jnp.int32) for _ in range(_NBUF)]
        + [pltpu.VMEM((_CHUNK,), jnp.int32) for _ in range(_NBUF)]
        + [pltpu.VMEM((_CHUNK, _HALF), jnp.float32) for _ in range(_NBUF)]
        + [pltpu.VMEM((_CHUNK,), jnp.int32) for _ in range(_NBUF // 2)]
        + [
            pltpu.VMEM((_CHUNK,), jnp.float32),
            pltpu.VMEM((_RPT * _G,), jnp.float32),
            pltpu.VMEM_SHARED((_N, _HALF), jnp.float32),
            pltpu.VMEM_SHARED((_N * _G,), jnp.float32),
            pltpu.SemaphoreType.DMA,
            pltpu.SemaphoreType.DMA,
            pltpu.SemaphoreType.DMA,
            pltpu.SemaphoreType.DMA,
            pltpu.SemaphoreType.DMA,
        ]
    )
    fn = pl.kernel(
        _sc_body,
        mesh=mesh,
        compiler_params=pltpu.CompilerParams(use_tc_tiling_on_sc=False),
        out_type=[jax.ShapeDtypeStruct((_N, _HALF), jnp.float32),
                  jax.ShapeDtypeStruct((_N, _HALF), jnp.float32),
                  jax.ShapeDtypeStruct((_N * _G,), jnp.float32),
                  jax.ShapeDtypeStruct((_N * _G,), jnp.float32)],
        scratch_types=scratch,
    )
    return fn(x2, edge_index, batch)


def _tc_body(p0_r, p1_r, x_r, ct0_r, ct1_r, b3_r,
             w1rel_r, w1root_r, b1_r, w2rel_r, w2root_r, b2_r,
             out_r, C2_r, S2_r, NG_r):
    k = pl.program_id(0)
    nk = pl.num_programs(0)

    @pl.when(k == 0)
    def _():
        C2_r[...] = jnp.zeros_like(C2_r)
        S2_r[...] = jnp.zeros_like(S2_r)
        NG_r[...] = jnp.zeros_like(NG_r)

    dnt = (((1,), (1,)), ((), ()))
    agg = jnp.concatenate([p0_r[...], p1_r[...]], axis=1)
    h = jnp.maximum(
        lax.dot_general(agg, w1rel_r[...], dnt,
                        preferred_element_type=jnp.float32)
        + lax.dot_general(x_r[...], w1root_r[...], dnt,
                          preferred_element_type=jnp.float32)
        + b1_r[...], 0.0)

    ct = ct0_r[...] + ct1_r[...]
    C2_r[...] += lax.dot_general(ct, h, (((0,), (0,)), ((), ())),
                                 preferred_element_type=jnp.float32)

    rows = h.shape[0]
    gids = lax.broadcasted_iota(jnp.int32, (_G, rows), 0)
    mask = (b3_r[...].reshape(1, rows) == gids).astype(jnp.float32)
    S2_r[...] += jnp.dot(mask, h, preferred_element_type=jnp.float32)
    NG_r[...] += jnp.broadcast_to(jnp.sum(mask, axis=1, keepdims=True),
                                  NG_r.shape)

    @pl.when(k == nk - 1)
    def _():
        ng = NG_r[...][:, :1]
        ps = (lax.dot_general(C2_r[...], w2rel_r[...], dnt,
                              preferred_element_type=jnp.float32)
              + lax.dot_general(S2_r[...], w2root_r[...], dnt,
                                preferred_element_type=jnp.float32)
              + ng * b2_r[...])
        pooled = ps / jnp.maximum(ng, 1.0)
        m = jnp.max(pooled, axis=1, keepdims=True)
        sh = pooled - m
        out_r[...] = sh - jnp.log(jnp.sum(jnp.exp(sh), axis=1, keepdims=True))


_ROWS_BLK = 1000
_NBLK = _N // _ROWS_BLK


def _tc_pooled(p0, p1, x, ct0, ct1, batch3,
               w1rel, w1root, b1r, w2rel, w2root, b2r):
    return pl.pallas_call(
        _tc_body,
        grid=(_NBLK,),
        in_specs=[
            pl.BlockSpec((_ROWS_BLK, _HALF), lambda k: (k, 0)),
            pl.BlockSpec((_ROWS_BLK, _HALF), lambda k: (k, 0)),
            pl.BlockSpec((_ROWS_BLK, _IN), lambda k: (k, 0)),
            pl.BlockSpec((_ROWS_BLK, _G), lambda k: (k, 0)),
            pl.BlockSpec((_ROWS_BLK, _G), lambda k: (k, 0)),
            pl.BlockSpec((1, 1, _ROWS_BLK), lambda k: (k, 0, 0)),
            pl.BlockSpec((_HID, _IN), lambda k: (0, 0)),
            pl.BlockSpec((_HID, _IN), lambda k: (0, 0)),
            pl.BlockSpec((1, _HID), lambda k: (0, 0)),
            pl.BlockSpec((_OUT, _HID), lambda k: (0, 0)),
            pl.BlockSpec((_OUT, _HID), lambda k: (0, 0)),
            pl.BlockSpec((1, _OUT), lambda k: (0, 0)),
        ],
        out_specs=pl.BlockSpec((_G, _OUT), lambda k: (0, 0)),
        out_shape=jax.ShapeDtypeStruct((_G, _OUT), jnp.float32),
        scratch_shapes=[
            pltpu.VMEM((_G, _HID), jnp.float32),
            pltpu.VMEM((_G, _HID), jnp.float32),
            pltpu.VMEM((_G, _OUT), jnp.float32),
        ],
    )(p0, p1, x, ct0, ct1, batch3,
      w1rel, w1root, b1r, w2rel, w2root, b2r)


def kernel(x, edge_index, batch, W1_rel, W1_root, b1, W2_rel, W2_root, b2):
    acc0, acc1, cp0, cp1 = _sc_scatter(x.reshape(2 * _N, _HALF),
                                       edge_index, batch)
    batch3 = batch.reshape(_NBLK, 1, _ROWS_BLK)
    return _tc_pooled(
        acc0, acc1, x, cp0.reshape(_N, _G), cp1.reshape(_N, _G), batch3,
        W1_rel, W1_root, b1.reshape(1, _HID),
        W2_rel, W2_root, b2.reshape(1, _OUT))


# speedup vs baseline: 1.0369x; 1.0160x over previous
import functools

import jax
import jax.numpy as jnp
from jax import lax
from jax.experimental import pallas as pl
from jax.experimental.pallas import tpu as pltpu
from jax.experimental.pallas import tpu_sc as plsc

_N = 10000
_E = 320000
_IN = 128
_HALF = _IN // 2
_HID = 256
_OUT = 128
_G = 16

_CHUNK = 128
_NCHUNK = _E // _CHUNK
_NSUB = 16
_BASE_ITERS = _NCHUNK // _NSUB
_XTRA = _NCHUNK - _BASE_ITERS * _NSUB
_NBUF = 8
_TILES = 16
_RPT = 624
_RTAIL = _N - _TILES * _RPT


def _sc_body(x2_hbm, edge_hbm, batch_hbm,
             acc0_hbm, acc1_hbm, cp0_hbm, cp1_hbm, *sc):
    srcs = sc[0:_NBUF]
    srcs2 = sc[_NBUF:2 * _NBUF]
    dsts = sc[2 * _NBUF:3 * _NBUF]
    gbs = sc[3 * _NBUF:4 * _NBUF]
    rows = sc[4 * _NBUF:5 * _NBUF]
    fidx = sc[5 * _NBUF:5 * _NBUF + _NBUF // 2]
    k = 5 * _NBUF + _NBUF // 2
    ones_v = sc[k]
    cob_v = sc[k + 1]
    acc_sh = sc[k + 2]
    c_sh = sc[k + 3]
    sem_i, sem_g, sem_b, sem_s, sem_c = sc[k + 4:k + 9]

    c = lax.axis_index("c")
    s = lax.axis_index("s")
    t = s
    rbase = pl.multiple_of(t * _RPT, 8)
    tail = pl.multiple_of(_TILES * _RPT, 8)

    fbase = pl.multiple_of(t * _RPT * _G, 128)
    flen = _RPT * _G
    ftail = pl.multiple_of(_TILES * _RPT * _G, 128)

    zero16 = jnp.zeros((16,), jnp.float32)
    one16 = jnp.ones((16,), jnp.float32)

    def zrow(i, carry):
        for j in range(_HALF // 16):
            rows[0][i, pl.ds(j * 16, 16)] = zero16
        return carry
    lax.fori_loop(0, _CHUNK, zrow, 0)

    def zco(i, carry):
        cob_v[pl.ds(i * 16, 16)] = zero16
        return carry
    lax.fori_loop(0, flen // 16, zco, 0)

    for j in range(_CHUNK // 16):
        ones_v[pl.ds(j * 16, 16)] = one16

    for k2 in range(4):
        pltpu.sync_copy(rows[0], acc_sh.at[pl.ds(rbase + k2 * _CHUNK, _CHUNK), :])
    pltpu.sync_copy(rows[0].at[pl.ds(0, _RPT - 512), :],
                    acc_sh.at[pl.ds(rbase + 512, _RPT - 512), :])
    pltpu.sync_copy(cob_v, c_sh.at[pl.ds(fbase, flen)])

    @pl.when(t == 0)
    def _():
        pltpu.sync_copy(rows[0].at[pl.ds(0, _RTAIL), :],
                        acc_sh.at[pl.ds(tail, _RTAIL), :])
        pltpu.sync_copy(cob_v.at[pl.ds(0, _RTAIL * _G)],
                        c_sh.at[pl.ds(ftail, _RTAIL * _G)])

    plsc.subcore_barrier()

    start = s * _BASE_ITERS + jnp.minimum(s, _XTRA)
    cnt = jnp.where(s < _XTRA, _BASE_ITERS + 1, _BASE_ITERS)
    nfull = cnt // _NBUF

    def cidx(b):
        f = fidx[b // 2]
        for j in range(_CHUNK // 16):
            sl = pl.ds(j * 16, 16)
            f[sl] = srcs[b][sl] * _G + gbs[b][sl]

    def counts(slots):
        chs = []
        for b in slots:
            cidx(b)
            chs.append(pltpu.async_copy(ones_v, c_sh.at[fidx[b // 2]], sem_c,
                                        add=True))
        for h in chs:
            h.wait()

    def idx2(b):
        for j in range(_CHUNK // 16):
            sl = pl.ds(j * 16, 16)
            srcs2[b][sl] = srcs[b][sl] * 2 + c

    def body(q, carry):
        g0 = start + q * _NBUF
        ihs = []
        for b in range(_NBUF):
            base = pl.multiple_of((g0 + b) * _CHUNK, 8)
            ihs.append(pltpu.async_copy(edge_hbm.at[0, pl.ds(base, _CHUNK)],
                                        srcs[b], sem_i))
            ihs.append(pltpu.async_copy(edge_hbm.at[1, pl.ds(base, _CHUNK)],
                                        dsts[b], sem_i))
        ghs, bhs = [], []
        for b in range(_NBUF):
            ihs[2 * b].wait()
            ihs[2 * b + 1].wait()
            idx2(b)
            ghs.append(pltpu.async_copy(x2_hbm.at[srcs2[b]], rows[b], sem_g))
            bhs.append(pltpu.async_copy(batch_hbm.at[dsts[b]], gbs[b], sem_b))
        shs = []
        for b in range(_NBUF):
            ghs[b].wait()
            shs.append(pltpu.async_copy(rows[b], acc_sh.at[dsts[b]], sem_s,
                                        add=True))
        for b in range(_NBUF):
            bhs[b].wait()

        @pl.when(c == 0)
        def _():
            counts(range(0, _NBUF, 2))

        @pl.when(c == 1)
        def _():
            counts(range(1, _NBUF, 2))

        for h in shs:
            h.wait()
        return carry

    lax.fori_loop(0, nfull, body, 0)

    def tail_body(i, carry):
        base = pl.multiple_of((start + i) * _CHUNK, 8)
        pltpu.sync_copy(edge_hbm.at[0, pl.ds(base, _CHUNK)], srcs[0])
        pltpu.sync_copy(edge_hbm.at[1, pl.ds(base, _CHUNK)], dsts[0])
        idx2(0)
        pltpu.async_copy(x2_hbm.at[srcs2[0]], rows[0], sem_g).wait()
        pltpu.sync_copy(rows[0], acc_sh.at[dsts[0]], add=True)

        @pl.when(lax.rem(i, 2) == c)
        def _():
            pltpu.async_copy(batch_hbm.at[dsts[0]], gbs[0], sem_b).wait()
            cidx(0)
            pltpu.sync_copy(ones_v, c_sh.at[fidx[0]], add=True)

        return carry

    lax.fori_loop(nfull * _NBUF, cnt, tail_body, 0)

    plsc.subcore_barrier()

    def wout(buf, sl, h0, h1):
        @pl.when(c == 0)
        def _():
            pltpu.sync_copy(buf, h0.at[sl, :])

        @pl.when(c == 1)
        def _():
            pltpu.sync_copy(buf, h1.at[sl, :])

    def wout1(buf, sl, h0, h1):
        @pl.when(c == 0)
        def _():
            pltpu.sync_copy(buf, h0.at[sl])

        @pl.when(c == 1)
        def _():
            pltpu.sync_copy(buf, h1.at[sl])

    for k2 in range(4):
        sl = pl.ds(rbase + k2 * _CHUNK, _CHUNK)
        pltpu.sync_copy(acc_sh.at[sl, :], rows[0])
        wout(rows[0], sl, acc0_hbm, acc1_hbm)
    sl = pl.ds(rbase + 512, _RPT - 512)
    pltpu.sync_copy(acc_sh.at[sl, :], rows[0].at[pl.ds(0, _RPT - 512), :])
    wout(rows[0].at[pl.ds(0, _RPT - 512), :], sl, acc0_hbm, acc1_hbm)
    pltpu.sync_copy(c_sh.at[pl.ds(fbase, flen)], cob_v)
    wout1(cob_v, pl.ds(fbase, flen), cp0_hbm, cp1_hbm)

    @pl.when(t == 0)
    def _():
        sl = pl.ds(tail, _RTAIL)
        pltpu.sync_copy(acc_sh.at[sl, :], rows[0].at[pl.ds(0, _RTAIL), :])
        wout(rows[0].at[pl.ds(0, _RTAIL), :], sl, acc0_hbm, acc1_hbm)
        pltpu.sync_copy(c_sh.at[pl.ds(ftail, _RTAIL * _G)],
                        cob_v.at[pl.ds(0, _RTAIL * _G)])
        wout1(cob_v.at[pl.ds(0, _RTAIL * _G)], pl.ds(ftail, _RTAIL * _G),
              cp0_hbm, cp1_hbm)


def _sc_scatter(x2, edge_index, batch):
    mesh = plsc.VectorSubcoreMesh(core_axis_name="c", subcore_axis_name="s")
    scratch = (
        [pltpu.VMEM((_CHUNK,), jnp.int32) for _ in range(_NBUF)]
        + [pltpu.VMEM((_CHUNK,), jnp.int32) for _ in range(_NBUF)]
        + [pltpu.VMEM((_CHUNK,), jnp.int32) for _ in range(_NBUF)]
        + [pltpu.VMEM((_CHUNK,), jnp.int32) for _ in range(_NBUF)]
        + [pltpu.VMEM((_CHUNK, _HALF), jnp.float32) for _ in range(_NBUF)]
        + [pltpu.VMEM((_CHUNK,), jnp.int32) for _ in range(_NBUF // 2)]
        + [
            pltpu.VMEM((_CHUNK,), jnp.float32),
            pltpu.VMEM((_RPT * _G,), jnp.float32),
            pltpu.VMEM_SHARED((_N, _HALF), jnp.float32),
            pltpu.VMEM_SHARED((_N * _G,), jnp.float32),
            pltpu.SemaphoreType.DMA,
            pltpu.SemaphoreType.DMA,
            pltpu.SemaphoreType.DMA,
            pltpu.SemaphoreType.DMA,
            pltpu.SemaphoreType.DMA,
        ]
    )
    fn = pl.kernel(
        _sc_body,
        mesh=mesh,
        compiler_params=pltpu.CompilerParams(use_tc_tiling_on_sc=False),
        out_type=[jax.ShapeDtypeStruct((_N, _HALF), jnp.float32),
                  jax.ShapeDtypeStruct((_N, _HALF), jnp.float32),
                  jax.ShapeDtypeStruct((_N * _G,), jnp.float32),
                  jax.ShapeDtypeStruct((_N * _G,), jnp.float32)],
        scratch_types=scratch,
    )
    return fn(x2, edge_index, batch)


def _tc_body(p0_r, p1_r, x_r, ct0_r, ct1_r, b3_r,
             w1rel_r, w1root_r, b1_r, w2rel_r, w2root_r, b2_r,
             out_r, C2_r, S2_r, NG_r):
    k = pl.program_id(0)
    nk = pl.num_programs(0)

    @pl.when(k == 0)
    def _():
        C2_r[...] = jnp.zeros_like(C2_r)
        S2_r[...] = jnp.zeros_like(S2_r)
        NG_r[...] = jnp.zeros_like(NG_r)

    dnt = (((1,), (1,)), ((), ()))
    agg = jnp.concatenate([p0_r[...], p1_r[...]], axis=1)
    h = jnp.maximum(
        lax.dot_general(agg, w1rel_r[...], dnt,
                        preferred_element_type=jnp.float32)
        + lax.dot_general(x_r[...], w1root_r[...], dnt,
                          preferred_element_type=jnp.float32)
        + b1_r[...], 0.0)

    ct = ct0_r[...] + ct1_r[...]
    C2_r[...] += lax.dot_general(ct, h, (((0,), (0,)), ((), ())),
                                 preferred_element_type=jnp.float32)

    rows = h.shape[0]
    gids = lax.broadcasted_iota(jnp.int32, (_G, rows), 0)
    mask = (b3_r[...].reshape(1, rows) == gids).astype(jnp.float32)
    S2_r[...] += jnp.dot(mask, h, preferred_element_type=jnp.float32)
    NG_r[...] += jnp.broadcast_to(jnp.sum(mask, axis=1, keepdims=True),
                                  NG_r.shape)

    @pl.when(k == nk - 1)
    def _():
        ng = NG_r[...][:, :1]
        ps = (lax.dot_general(C2_r[...], w2rel_r[...], dnt,
                              preferred_element_type=jnp.float32)
              + lax.dot_general(S2_r[...], w2root_r[...], dnt,
                                preferred_element_type=jnp.float32)
              + ng * b2_r[...])
        pooled = ps / jnp.maximum(ng, 1.0)
        m = jnp.max(pooled, axis=1, keepdims=True)
        sh = pooled - m
        out_r[...] = sh - jnp.log(jnp.sum(jnp.exp(sh), axis=1, keepdims=True))


_ROWS_BLK = 1000
_NBLK = _N // _ROWS_BLK


def _tc_pooled(p0, p1, x, ct0, ct1, batch3,
               w1rel, w1root, b1r, w2rel, w2root, b2r):
    return pl.pallas_call(
        _tc_body,
        grid=(_NBLK,),
        in_specs=[
            pl.BlockSpec((_ROWS_BLK, _HALF), lambda k: (k, 0)),
            pl.BlockSpec((_ROWS_BLK, _HALF), lambda k: (k, 0)),
            pl.BlockSpec((_ROWS_BLK, _IN), lambda k: (k, 0)),
            pl.BlockSpec((_ROWS_BLK, _G), lambda k: (k, 0)),
            pl.BlockSpec((_ROWS_BLK, _G), lambda k: (k, 0)),
            pl.BlockSpec((1, 1, _ROWS_BLK), lambda k: (k, 0, 0)),
            pl.BlockSpec((_HID, _IN), lambda k: (0, 0)),
            pl.BlockSpec((_HID, _IN), lambda k: (0, 0)),
            pl.BlockSpec((1, _HID), lambda k: (0, 0)),
            pl.BlockSpec((_OUT, _HID), lambda k: (0, 0)),
            pl.BlockSpec((_OUT, _HID), lambda k: (0, 0)),
            pl.BlockSpec((1, _OUT), lambda k: (0, 0)),
        ],
        out_specs=pl.BlockSpec((_G, _OUT), lambda k: (0, 0)),
        out_shape=jax.ShapeDtypeStruct((_G, _OUT), jnp.float32),
        scratch_shapes=[
            pltpu.VMEM((_G, _HID), jnp.float32),
            pltpu.VMEM((_G, _HID), jnp.float32),
            pltpu.VMEM((_G, _OUT), jnp.float32),
        ],
    )(p0, p1, x, ct0, ct1, batch3,
      w1rel, w1root, b1r, w2rel, w2root, b2r)


def kernel(x, edge_index, batch, W1_rel, W1_root, b1, W2_rel, W2_root, b2):
    acc0, acc1, cp0, cp1 = _sc_scatter(x.reshape(2 * _N, _HALF),
                                       edge_index, batch)
    batch3 = batch.reshape(_NBLK, 1, _ROWS_BLK)
    return _tc_pooled(
        acc0, acc1, x, cp0.reshape(_N, _G), cp1.reshape(_N, _G), batch3,
        W1_rel, W1_root, b1.reshape(1, _HID),
        W2_rel, W2_root, b2.reshape(1, _OUT))


# speedup vs baseline: 1.0406x; 1.0036x over previous
import functools

import jax
import jax.numpy as jnp
from jax import lax
from jax.experimental import pallas as pl
from jax.experimental.pallas import tpu as pltpu
from jax.experimental.pallas import tpu_sc as plsc

_N = 10000
_E = 320000
_IN = 128
_HALF = _IN // 2
_HID = 256
_OUT = 128
_G = 16

_CHUNK = 128
_NCHUNK = _E // _CHUNK
_NSUB = 16
_BASE_ITERS = _NCHUNK // _NSUB
_XTRA = _NCHUNK - _BASE_ITERS * _NSUB
_NBUF = 8
_TILES = 16
_RPT = 624
_RTAIL = _N - _TILES * _RPT


def _sc_body(x2_hbm, edge_hbm, batch_hbm,
             acc0_hbm, acc1_hbm, cp0_hbm, cp1_hbm, *sc):
    srcs = sc[0:_NBUF]
    srcs2 = sc[_NBUF:2 * _NBUF]
    dsts = sc[2 * _NBUF:3 * _NBUF]
    gbs = sc[3 * _NBUF:4 * _NBUF]
    rows = sc[4 * _NBUF:5 * _NBUF]
    fidx = sc[5 * _NBUF:5 * _NBUF + _NBUF // 2]
    k = 5 * _NBUF + _NBUF // 2
    ones_v = sc[k]
    cob_v = sc[k + 1]
    acc_sh = sc[k + 2]
    c_sh = sc[k + 3]
    sem_i, sem_g, sem_b, sem_s, sem_c = sc[k + 4:k + 9]

    c = lax.axis_index("c")
    s = lax.axis_index("s")
    t = s
    rbase = pl.multiple_of(t * _RPT, 8)
    tail = pl.multiple_of(_TILES * _RPT, 8)

    fbase = pl.multiple_of(t * _RPT * _G, 128)
    flen = _RPT * _G
    ftail = pl.multiple_of(_TILES * _RPT * _G, 128)

    zero16 = jnp.zeros((16,), jnp.float32)
    one16 = jnp.ones((16,), jnp.float32)

    def zrow(i, carry):
        for j in range(_HALF // 16):
            rows[0][i, pl.ds(j * 16, 16)] = zero16
        return carry
    lax.fori_loop(0, _CHUNK, zrow, 0)

    def zco(i, carry):
        cob_v[pl.ds(i * 16, 16)] = zero16
        return carry
    lax.fori_loop(0, flen // 16, zco, 0)

    for j in range(_CHUNK // 16):
        ones_v[pl.ds(j * 16, 16)] = one16

    for k2 in range(4):
        pltpu.sync_copy(rows[0], acc_sh.at[pl.ds(rbase + k2 * _CHUNK, _CHUNK), :])
    pltpu.sync_copy(rows[0].at[pl.ds(0, _RPT - 512), :],
                    acc_sh.at[pl.ds(rbase + 512, _RPT - 512), :])
    pltpu.sync_copy(cob_v, c_sh.at[pl.ds(fbase, flen)])

    @pl.when(t == 0)
    def _():
        pltpu.sync_copy(rows[0].at[pl.ds(0, _RTAIL), :],
                        acc_sh.at[pl.ds(tail, _RTAIL), :])
        pltpu.sync_copy(cob_v.at[pl.ds(0, _RTAIL * _G)],
                        c_sh.at[pl.ds(ftail, _RTAIL * _G)])

    plsc.subcore_barrier()

    start = s * _BASE_ITERS + jnp.minimum(s, _XTRA)
    cnt = jnp.where(s < _XTRA, _BASE_ITERS + 1, _BASE_ITERS)
    nfull = cnt // _NBUF

    def cidx(b):
        f = fidx[b // 2]
        for j in range(_CHUNK // 16):
            sl = pl.ds(j * 16, 16)
            f[sl] = srcs[b][sl] * _G + gbs[b][sl]

    def counts(slots):
        chs = []
        for b in slots:
            cidx(b)
            chs.append(pltpu.async_copy(ones_v, c_sh.at[fidx[b // 2]], sem_c,
                                        add=True))
        for h in chs:
            h.wait()

    def idx2(b):
        for j in range(_CHUNK // 16):
            sl = pl.ds(j * 16, 16)
            srcs2[b][sl] = srcs[b][sl] * 2 + c

    def body(q, carry):
        g0 = start + q * _NBUF
        ihs = []
        for b in range(_NBUF):
            base = pl.multiple_of((g0 + b) * _CHUNK, 8)
            ihs.append(pltpu.async_copy(edge_hbm.at[0, pl.ds(base, _CHUNK)],
                                        srcs[b], sem_i))
            ihs.append(pltpu.async_copy(edge_hbm.at[1, pl.ds(base, _CHUNK)],
                                        dsts[b], sem_i))
        ghs, bhs = [], []
        for b in range(_NBUF):
            ihs[2 * b].wait()
            ihs[2 * b + 1].wait()
            idx2(b)
            ghs.append(pltpu.async_copy(x2_hbm.at[srcs2[b]], rows[b], sem_g))
            bhs.append(pltpu.async_copy(batch_hbm.at[dsts[b]], gbs[b], sem_b))
        shs = []
        for b in range(_NBUF):
            ghs[b].wait()
            shs.append(pltpu.async_copy(rows[b], acc_sh.at[dsts[b]], sem_s,
                                        add=True))
        for b in range(_NBUF):
            bhs[b].wait()

        @pl.when(c == 0)
        def _():
            counts(range(0, _NBUF, 2))

        @pl.when(c == 1)
        def _():
            counts(range(1, _NBUF, 2))

        for h in shs:
            h.wait()
        return carry

    lax.fori_loop(0, nfull, body, 0)

    def tail_body(i, carry):
        base = pl.multiple_of((start + i) * _CHUNK, 8)
        pltpu.sync_copy(edge_hbm.at[0, pl.ds(base, _CHUNK)], srcs[0])
        pltpu.sync_copy(edge_hbm.at[1, pl.ds(base, _CHUNK)], dsts[0])
        idx2(0)
        pltpu.async_copy(x2_hbm.at[srcs2[0]], rows[0], sem_g).wait()
        pltpu.sync_copy(rows[0], acc_sh.at[dsts[0]], add=True)

        @pl.when(lax.rem(i, 2) == c)
        def _():
            pltpu.async_copy(batch_hbm.at[dsts[0]], gbs[0], sem_b).wait()
            cidx(0)
            pltpu.sync_copy(ones_v, c_sh.at[fidx[0]], add=True)

        return carry

    lax.fori_loop(nfull * _NBUF, cnt, tail_body, 0)

    plsc.subcore_barrier()

    def wout(buf, sl, h0, h1):
        @pl.when(c == 0)
        def _():
            pltpu.sync_copy(buf, h0.at[sl, :])

        @pl.when(c == 1)
        def _():
            pltpu.sync_copy(buf, h1.at[sl, :])

    def wout1(buf, sl, h0, h1):
        @pl.when(c == 0)
        def _():
            pltpu.sync_copy(buf, h0.at[sl])

        @pl.when(c == 1)
        def _():
            pltpu.sync_copy(buf, h1.at[sl])

    for k2 in range(4):
        sl = pl.ds(rbase + k2 * _CHUNK, _CHUNK)
        pltpu.sync_copy(acc_sh.at[sl, :], rows[0])
        wout(rows[0], sl, acc0_hbm, acc1_hbm)
    sl = pl.ds(rbase + 512, _RPT - 512)
    pltpu.sync_copy(acc_sh.at[sl, :], rows[0].at[pl.ds(0, _RPT - 512), :])
    wout(rows[0].at[pl.ds(0, _RPT - 512), :], sl, acc0_hbm, acc1_hbm)
    pltpu.sync_copy(c_sh.at[pl.ds(fbase, flen)], cob_v)
    wout1(cob_v, pl.ds(fbase, flen), cp0_hbm, cp1_hbm)

    @pl.when(t == 0)
    def _():
        sl = pl.ds(tail, _RTAIL)
        pltpu.sync_copy(acc_sh.at[sl, :], rows[0].at[pl.ds(0, _RTAIL), :])
        wout(rows[0].at[pl.ds(0, _RTAIL), :], sl, acc0_hbm, acc1_hbm)
        pltpu.sync_copy(c_sh.at[pl.ds(ftail, _RTAIL * _G)],
                        cob_v.at[pl.ds(0, _RTAIL * _G)])
        wout1(cob_v.at[pl.ds(0, _RTAIL * _G)], pl.ds(ftail, _RTAIL * _G),
              cp0_hbm, cp1_hbm)


def _sc_scatter(x2, edge_index, batch):
    mesh = plsc.VectorSubcoreMesh(core_axis_name="c", subcore_axis_name="s")
    scratch = (
        [pltpu.VMEM((_CHUNK,), jnp.int32) for _ in range(_NBUF)]
        + [pltpu.VMEM((_CHUNK,), jnp.int32) for _ in range(_NBUF)]
        + [pltpu.VMEM((_CHUNK,), jnp.int32) for _ in range(_NBUF)]
        + [pltpu.VMEM((_CHUNK,), jnp.int32) for _ in range(_NBUF)]
        + [pltpu.VMEM((_CHUNK, _HALF), jnp.float32) for _ in range(_NBUF)]
        + [pltpu.VMEM((_CHUNK,), jnp.int32) for _ in range(_NBUF // 2)]
        + [
            pltpu.VMEM((_CHUNK,), jnp.float32),
            pltpu.VMEM((_RPT * _G,), jnp.float32),
            pltpu.VMEM_SHARED((_N, _HALF), jnp.float32),
            pltpu.VMEM_SHARED((_N * _G,), jnp.float32),
            pltpu.SemaphoreType.DMA,
            pltpu.SemaphoreType.DMA,
            pltpu.SemaphoreType.DMA,
            pltpu.SemaphoreType.DMA,
            pltpu.SemaphoreType.DMA,
        ]
    )
    fn = pl.kernel(
        _sc_body,
        mesh=mesh,
        compiler_params=pltpu.CompilerParams(use_tc_tiling_on_sc=False),
        out_type=[jax.ShapeDtypeStruct((_N, _HALF), jnp.float32),
                  jax.ShapeDtypeStruct((_N, _HALF), jnp.float32),
                  jax.ShapeDtypeStruct((_N * _G,), jnp.float32),
                  jax.ShapeDtypeStruct((_N * _G,), jnp.float32)],
        scratch_types=scratch,
    )
    return fn(x2, edge_index, batch)


def _tc_body(p0_r, p1_r, x_r, ct0_r, ct1_r, b3_r,
             w1rel_r, w1root_r, b1_r, w2rel_r, w2root_r, b2_r,
             out_r, C2_r, S2_r, NG_r):
    k = pl.program_id(0)
    nk = pl.num_programs(0)

    @pl.when(k == 0)
    def _():
        C2_r[...] = jnp.zeros_like(C2_r)
        S2_r[...] = jnp.zeros_like(S2_r)
        NG_r[...] = jnp.zeros_like(NG_r)

    dnt = (((1,), (1,)), ((), ()))
    agg = jnp.concatenate([p0_r[...], p1_r[...]], axis=1)
    h = jnp.maximum(
        lax.dot_general(agg, w1rel_r[...], dnt,
                        preferred_element_type=jnp.float32)
        + lax.dot_general(x_r[...], w1root_r[...], dnt,
                          preferred_element_type=jnp.float32)
        + b1_r[...], 0.0)

    ct = ct0_r[...] + ct1_r[...]
    C2_r[...] += lax.dot_general(ct, h, (((0,), (0,)), ((), ())),
                                 preferred_element_type=jnp.float32)

    rows = h.shape[0]
    gids = lax.broadcasted_iota(jnp.int32, (_G, rows), 0)
    mask = (b3_r[...].reshape(1, rows) == gids).astype(jnp.float32)
    S2_r[...] += jnp.dot(mask, h, preferred_element_type=jnp.float32)
    NG_r[...] += jnp.broadcast_to(jnp.sum(mask, axis=1, keepdims=True),
                                  NG_r.shape)

    @pl.when(k == nk - 1)
    def _():
        ng = NG_r[...][:, :1]
        ps = (lax.dot_general(C2_r[...], w2rel_r[...], dnt,
                              preferred_element_type=jnp.float32)
              + lax.dot_general(S2_r[...], w2root_r[...], dnt,
                                preferred_element_type=jnp.float32)
              + ng * b2_r[...])
        pooled = ps / jnp.maximum(ng, 1.0)
        m = jnp.max(pooled, axis=1, keepdims=True)
        sh = pooled - m
        out_r[...] = sh - jnp.log(jnp.sum(jnp.exp(sh), axis=1, keepdims=True))


_ROWS_BLK = 10000
_NBLK = _N // _ROWS_BLK


def _tc_pooled(p0, p1, x, ct0, ct1, batch3,
               w1rel, w1root, b1r, w2rel, w2root, b2r):
    return pl.pallas_call(
        _tc_body,
        grid=(_NBLK,),
        in_specs=[
            pl.BlockSpec((_ROWS_BLK, _HALF), lambda k: (k, 0)),
            pl.BlockSpec((_ROWS_BLK, _HALF), lambda k: (k, 0)),
            pl.BlockSpec((_ROWS_BLK, _IN), lambda k: (k, 0)),
            pl.BlockSpec((_ROWS_BLK, _G), lambda k: (k, 0)),
            pl.BlockSpec((_ROWS_BLK, _G), lambda k: (k, 0)),
            pl.BlockSpec((1, 1, _ROWS_BLK), lambda k: (k, 0, 0)),
            pl.BlockSpec((_HID, _IN), lambda k: (0, 0)),
            pl.BlockSpec((_HID, _IN), lambda k: (0, 0)),
            pl.BlockSpec((1, _HID), lambda k: (0, 0)),
            pl.BlockSpec((_OUT, _HID), lambda k: (0, 0)),
            pl.BlockSpec((_OUT, _HID), lambda k: (0, 0)),
            pl.BlockSpec((1, _OUT), lambda k: (0, 0)),
        ],
        out_specs=pl.BlockSpec((_G, _OUT), lambda k: (0, 0)),
        out_shape=jax.ShapeDtypeStruct((_G, _OUT), jnp.float32),
        scratch_shapes=[
            pltpu.VMEM((_G, _HID), jnp.float32),
            pltpu.VMEM((_G, _HID), jnp.float32),
            pltpu.VMEM((_G, _OUT), jnp.float32),
        ],
    )(p0, p1, x, ct0, ct1, batch3,
      w1rel, w1root, b1r, w2rel, w2root, b2r)


def kernel(x, edge_index, batch, W1_rel, W1_root, b1, W2_rel, W2_root, b2):
    acc0, acc1, cp0, cp1 = _sc_scatter(x.reshape(2 * _N, _HALF),
                                       edge_index, batch)
    batch3 = batch.reshape(_NBLK, 1, _ROWS_BLK)
    return _tc_pooled(
        acc0, acc1, x, cp0.reshape(_N, _G), cp1.reshape(_N, _G), batch3,
        W1_rel, W1_root, b1.reshape(1, _HID),
        W2_rel, W2_root, b2.reshape(1, _OUT))
